# baseline (speedup 1.0000x reference)
import jax
import jax.numpy as jnp
from jax.experimental import pallas as pl

_NUM_SELECTS = 8
_BLOCK_TOKENS = 2048


def _gate_kernel(x_ref, w_ref, out_ref):
    x = x_ref[...]
    w = w_ref[...]
    logits = jax.lax.dot_general(
        x, w, (((1,), (1,)), ((), ())), preferred_element_type=jnp.float32
    )
    neg_inf = jnp.float32(-jnp.inf)
    work = logits
    for _ in range(_NUM_SELECTS - 1):
        mx = jnp.max(work, axis=1, keepdims=True)
        work = jnp.where(work == mx, neg_inf, work)
    t = jnp.max(work, axis=1, keepdims=True)
    out_ref[...] = jnp.where(logits >= t, jnp.float32(1.0), jnp.float32(0.0))


def kernel(routing_inputs, w_gate):
    num_tokens, hidden = routing_inputs.shape
    num_experts = w_gate.shape[0]
    bt = min(_BLOCK_TOKENS, num_tokens)
    grid = (num_tokens // bt,)
    return pl.pallas_call(
        _gate_kernel,
        grid=grid,
        in_specs=[
            pl.BlockSpec((bt, hidden), lambda i: (i, 0)),
            pl.BlockSpec((num_experts, hidden), lambda i: (0, 0)),
        ],
        out_specs=pl.BlockSpec((bt, num_experts), lambda i: (i, 0)),
        out_shape=jax.ShapeDtypeStruct((num_tokens, num_experts), jnp.float32),
    )(routing_inputs, w_gate)


# speedup vs baseline: 1.1878x; 1.1878x over previous
import jax
import jax.numpy as jnp
from jax.experimental import pallas as pl

_NUM_SELECTS = 8
_BLOCK_TOKENS = 4096


def _gate_kernel(x_ref, w_ref, out_ref):
    x = x_ref[...]
    w = w_ref[...]
    dims = (((1,), (1,)), ((), ()))
    logits_t = jax.lax.dot_general(
        w, x, dims, preferred_element_type=jnp.float32
    )
    neg_inf = jnp.float32(-jnp.inf)
    work = logits_t
    for _ in range(_NUM_SELECTS - 1):
        mx = jnp.max(work, axis=0, keepdims=True)
        work = jnp.where(work == mx, neg_inf, work)
    t = jnp.max(work, axis=0, keepdims=True)
    mask_t = jnp.where(logits_t >= t, jnp.float32(1.0), jnp.float32(0.0))
    out_ref[...] = jnp.transpose(mask_t)


def kernel(routing_inputs, w_gate):
    num_tokens, hidden = routing_inputs.shape
    num_experts = w_gate.shape[0]
    bt = min(_BLOCK_TOKENS, num_tokens)
    grid = (num_tokens // bt,)
    return pl.pallas_call(
        _gate_kernel,
        grid=grid,
        in_specs=[
            pl.BlockSpec((bt, hidden), lambda i: (i, 0)),
            pl.BlockSpec((num_experts, hidden), lambda i: (0, 0)),
        ],
        out_specs=pl.BlockSpec((bt, num_experts), lambda i: (i, 0)),
        out_shape=jax.ShapeDtypeStruct((num_tokens, num_experts), jnp.float32),
    )(routing_inputs, w_gate)
